# Initial kernel scaffold; baseline (speedup 1.0000x reference)
import jax
import jax.numpy as jnp
from jax.experimental import pallas as pl


def kernel(batch_data, user_table, item_table, conf_edges_0, conf_edges_1, cond_edges_0, cond_edges_1):
    raise NotImplementedError("write your pallas kernel here")


# speedup vs baseline: 32.1421x; 32.1421x over previous
import functools

import jax
import jax.numpy as jnp
from jax import lax
from jax.experimental import pallas as pl
from jax.experimental.pallas import tpu as pltpu
from jax.experimental.pallas import tpu_sc as plsc

H = 16
NROWS = 100001
NPAD = 100352
PT = NPAD // 16
BLK = 896
NBLK = PT // BLK
LBLK = 224
LNBLK = PT // LBLK
IW = 128
GR = 8
ALPHA = 0.5
REG_WEIGHT = 0.001


def _mesh():
    return plsc.VectorSubcoreMesh(core_axis_name="c", subcore_axis_name="s")


def _f32(shape):
    return jax.ShapeDtypeStruct(shape, jnp.float32)


def _zero_rows(ref, n):
    def body(r, carry):
        ref[r, :] = jnp.zeros((H,), jnp.float32)
        return carry
    lax.fori_loop(0, n, body, 0)


def _lanesum(v):
    s = v[0]
    for r in range(1, 16):
        s = s + v[r]
    return s


def _rsqrt16(d):
    y = 1.0 / d
    for _ in range(20):
        y = y * (1.5 - 0.5 * d * y * y)
    return y


def _deg_prep(u2d, i2d, eu0, eu1, ei0, ei1):
    er = u2d.shape[0]
    cnt_r = er // 16
    ngr, rem = divmod(cnt_r, GR)

    out_type = [_f32((NPAD,))] * 2 + [_f32((NPAD, H))] * 4
    scratch = [
        pltpu.VMEM_SHARED((NPAD,), jnp.float32),
        pltpu.VMEM((GR, IW), jnp.int32),
        pltpu.VMEM((IW,), jnp.float32),
        pltpu.VMEM((BLK,), jnp.float32),
        pltpu.VMEM((BLK,), jnp.float32),
        pltpu.VMEM((BLK, H), jnp.float32),
        pltpu.VMEM((BLK, H), jnp.float32),
        pltpu.VMEM((BLK, H), jnp.float32),
        pltpu.VMEM((BLK, H), jnp.float32),
    ]

    @functools.partial(pl.kernel, mesh=_mesh(), out_type=out_type,
                       scratch_types=scratch,
                       compiler_params=pltpu.CompilerParams(
                           use_tc_tiling_on_sc=False))
    def k(u_ref, i_ref, eu0_ref, eu1_ref, ei0_ref, ei1_ref,
          dvu_ref, dvi_ref, tu0_ref, tu1_ref, ti0_ref, ti1_ref,
          deg_sh, idxb, onesb, degb, dvb, e0b, e1b, t0b, t1b):
        c = lax.axis_index("c")
        s = lax.axis_index("s")
        tb = s * PT

        for kk in range(IW // 16):
            onesb[pl.ds(kk * 16, 16)] = jnp.full((16,), 1.0, jnp.float32)

        def zb(kk, carry):
            degb[pl.ds(kk * 16, 16)] = jnp.zeros((16,), jnp.float32)
            return carry
        lax.fori_loop(0, BLK // 16, zb, 0)
        for b in range(NBLK):
            pltpu.sync_copy(degb, deg_sh.at[pl.ds(tb + b * BLK, BLK)])
        plsc.subcore_barrier()

        def scatter_side(idx2d):
            row0 = s * cnt_r

            def grp(g, carry):
                pltpu.sync_copy(idx2d.at[pl.ds(row0 + g * GR, GR)], idxb)
                for j in range(GR):
                    pltpu.sync_copy(onesb, deg_sh.at[idxb.at[j]], add=True)
                return carry
            lax.fori_loop(0, ngr, grp, 0)
            if rem:
                base = row0 + ngr * GR
                pltpu.sync_copy(idx2d.at[pl.ds(base, rem)],
                                idxb.at[pl.ds(0, rem)])
                for j in range(rem):
                    pltpu.sync_copy(onesb, deg_sh.at[idxb.at[j]], add=True)

        @pl.when(c == 0)
        def _():
            scatter_side(u_ref)

        @pl.when(c == 1)
        def _():
            scatter_side(i_ref)

        plsc.subcore_barrier()

        def drain_side(e0_ref, e1_ref, dv_ref, t0_ref, t1_ref):
            def drain(b, carry):
                off = pl.multiple_of(tb + b * BLK, BLK)
                blk = pl.ds(off, BLK)
                pltpu.sync_copy(deg_sh.at[blk], degb)

                def rsq(kk, carry):
                    sl = pl.ds(kk * 16, 16)
                    dg = degb[sl]
                    d = jnp.maximum(dg, 1.0)
                    y = _rsqrt16(d)
                    dvb[sl] = jnp.where(dg > 0.0, y,
                                        jnp.zeros((16,), jnp.float32))
                    return carry
                lax.fori_loop(0, BLK // 16, rsq, 0)
                pltpu.sync_copy(dvb, dv_ref.at[blk])
                pltpu.sync_copy(e0_ref.at[blk], e0b)
                pltpu.sync_copy(e1_ref.at[blk], e1b)

                def rowm(kk, carry):
                    dv16 = dvb[pl.ds(kk * 16, 16)]
                    for r in range(16):
                        row = kk * 16 + r
                        dv = dv16[r]
                        t0b[row, :] = e0b[row, :] * dv
                        t1b[row, :] = e1b[row, :] * dv
                    return carry
                lax.fori_loop(0, BLK // 16, rowm, 0)
                pltpu.sync_copy(t0b, t0_ref.at[blk])
                pltpu.sync_copy(t1b, t1_ref.at[blk])
                return carry
            lax.fori_loop(0, NBLK, drain, 0)

        @pl.when(c == 0)
        def _():
            drain_side(eu0_ref, eu1_ref, dvu_ref, tu0_ref, tu1_ref)

        @pl.when(c == 1)
        def _():
            drain_side(ei0_ref, ei1_ref, dvi_ref, ti0_ref, ti1_ref)

    return k(u2d, i2d, eu0, eu1, ei0, ei1)


def _layer(u2d, i2d, tu, ti, dvu, dvi, mode, extra=None):
    er = u2d.shape[0]
    cnt_r = er // 16
    ngr, rem = divmod(cnt_r, GR)

    n_out = 8 if mode == 1 else 4
    out_type = [_f32((NPAD, H))] * n_out
    scratch = [
        pltpu.VMEM_SHARED((NPAD, H), jnp.float32),
        pltpu.VMEM((GR, IW), jnp.int32),
        pltpu.VMEM((GR, IW), jnp.int32),
        pltpu.VMEM((GR * IW, H), jnp.float32),
        pltpu.VMEM((LBLK, H), jnp.float32),
        pltpu.VMEM((LBLK,), jnp.float32),
        pltpu.VMEM((LBLK, H), jnp.float32),
        pltpu.VMEM((LBLK, H), jnp.float32),
        pltpu.SemaphoreType.DMA,
    ]

    def body(refs):
        if mode == 1:
            (u_ref, i_ref, tu0, tu1, ti0, ti1, dvu_ref, dvi_ref,
             xu0, xu1, xi0, xi1, nu0, nu1, ni0, ni1,
             acc_sh, sidx, didx, rowsb, abuf, dvb, o1b, o2b, sem) = refs
        else:
            (u_ref, i_ref, tu0, tu1, ti0, ti1, dvu_ref, dvi_ref,
             eu0_ref, eu1_ref, ei0_ref, ei1_ref,
             xu0, xu1, xi0, xi1,
             au0, au1, ai0, ai1,
             acc_sh, sidx, didx, rowsb, abuf, dvb, o1b, o2b, sem) = refs

        c = lax.axis_index("c")
        s = lax.axis_index("s")
        tb = s * PT

        def edge_pass(src2d, dst2d, tsrc):
            row0 = s * cnt_r

            def grp_body(k_rows, gbase):
                pltpu.sync_copy(src2d.at[pl.ds(gbase, k_rows)],
                                sidx.at[pl.ds(0, k_rows)])
                pltpu.sync_copy(dst2d.at[pl.ds(gbase, k_rows)],
                                didx.at[pl.ds(0, k_rows)])
                descs = []
                for j in range(k_rows):
                    descs.append(pltpu.async_copy(
                        tsrc.at[sidx.at[j]],
                        rowsb.at[pl.ds(j * IW, IW)], sem))
                for dsc in descs:
                    dsc.wait()
                for j in range(k_rows):
                    pltpu.sync_copy(rowsb.at[pl.ds(j * IW, IW)],
                                    acc_sh.at[didx.at[j]], add=True)

            def grp(g, carry):
                grp_body(GR, row0 + g * GR)
                return carry
            lax.fori_loop(0, ngr, grp, 0)
            if rem:
                grp_body(rem, row0 + ngr * GR)

        def run_side(dst2d, src2d, tsrc_pair, dv_ref, outs):
            for h in range(2):
                _zero_rows(o1b, LBLK)

                def zblk(b, carry):
                    off = pl.multiple_of(tb + b * LBLK, LBLK)
                    pltpu.sync_copy(o1b, acc_sh.at[pl.ds(off, LBLK)])
                    return carry
                lax.fori_loop(0, LNBLK, zblk, 0)
                plsc.subcore_barrier()
                edge_pass(src2d, dst2d, tsrc_pair[h])
                plsc.subcore_barrier()

                if mode == 1:
                    xout, tout = outs[0][h], outs[1][h]

                    def drain(b, carry):
                        off = pl.multiple_of(tb + b * LBLK, LBLK)
                        blk = pl.ds(off, LBLK)
                        pltpu.sync_copy(acc_sh.at[blk], abuf)
                        pltpu.sync_copy(dv_ref.at[blk], dvb)

                        def rowm(kk, c2):
                            dv16 = dvb[pl.ds(kk * 16, 16)]
                            for r in range(16):
                                row = kk * 16 + r
                                dv = dv16[r]
                                x = abuf[row, :] * dv
                                o1b[row, :] = x
                                o2b[row, :] = x * dv
                            return c2
                        lax.fori_loop(0, LBLK // 16, rowm, 0)
                        pltpu.sync_copy(o1b, xout.at[blk])
                        pltpu.sync_copy(o2b, tout.at[blk])
                        return carry
                    lax.fori_loop(0, LNBLK, drain, 0)
                else:
                    e_ref, x_ref, aout = (outs[0][h], outs[1][h],
                                          outs[2][h])

                    def drain(b, carry):
                        off = pl.multiple_of(tb + b * LBLK, LBLK)
                        blk = pl.ds(off, LBLK)
                        pltpu.sync_copy(acc_sh.at[blk], abuf)
                        pltpu.sync_copy(dv_ref.at[blk], dvb)
                        pltpu.sync_copy(e_ref.at[blk], o1b)
                        pltpu.sync_copy(x_ref.at[blk], o2b)

                        def rowm(kk, c2):
                            dv16 = dvb[pl.ds(kk * 16, 16)]
                            for r in range(16):
                                row = kk * 16 + r
                                a = (o1b[row, :] + o2b[row, :]
                                     + abuf[row, :] * dv16[r]) * (1.0 / 3.0)
                                abuf[row, :] = a
                            return c2
                        lax.fori_loop(0, LBLK // 16, rowm, 0)
                        pltpu.sync_copy(abuf, aout.at[blk])
                        return carry
                    lax.fori_loop(0, LNBLK, drain, 0)

        if mode == 1:
            @pl.when(c == 0)
            def _():
                run_side(u_ref, i_ref, (ti0, ti1), dvu_ref,
                         ((xu0, xu1), (nu0, nu1)))

            @pl.when(c == 1)
            def _():
                run_side(i_ref, u_ref, (tu0, tu1), dvi_ref,
                         ((xi0, xi1), (ni0, ni1)))
        else:
            @pl.when(c == 0)
            def _():
                run_side(u_ref, i_ref, (ti0, ti1), dvu_ref,
                         ((eu0_ref, eu1_ref), (xu0, xu1), (au0, au1)))

            @pl.when(c == 1)
            def _():
                run_side(i_ref, u_ref, (tu0, tu1), dvi_ref,
                         ((ei0_ref, ei1_ref), (xi0, xi1), (ai0, ai1)))

    @functools.partial(pl.kernel, mesh=_mesh(), out_type=out_type,
                       scratch_types=scratch,
                       compiler_params=pltpu.CompilerParams(
                           use_tc_tiling_on_sc=False))
    def k(*refs):
        body(refs)

    if mode == 1:
        return k(u2d, i2d, tu[0], tu[1], ti[0], ti[1], dvu, dvi)
    return k(u2d, i2d, tu[0], tu[1], ti[0], ti[1], dvu, dvi, *extra)


def _score(ub, pb, nb, accs, eu0, eu1, ei0, ei1):
    B = 4096
    SSB = NPAD // 32
    SSBLK = 784
    NSSB = SSB // SSBLK

    flat_accs = []
    for g in range(2):
        for kind in range(2):
            flat_accs.extend(accs[g][kind])

    out_type = [_f32((B,))] * 8 + [_f32((32, H))] * 2
    scratch = [
        pltpu.VMEM((IW,), jnp.int32),
        pltpu.VMEM((IW,), jnp.int32),
        pltpu.VMEM((IW,), jnp.int32),
        pltpu.VMEM((IW, H), jnp.float32),
        pltpu.VMEM((IW, H), jnp.float32),
        pltpu.VMEM((IW, H), jnp.float32),
        pltpu.VMEM((IW, H), jnp.float32),
        pltpu.VMEM((IW, H), jnp.float32),
        pltpu.VMEM((IW, H), jnp.float32),
        pltpu.VMEM((IW,), jnp.float32),
        pltpu.VMEM((IW,), jnp.float32),
        pltpu.VMEM((SSBLK, H), jnp.float32),
        pltpu.VMEM((16,), jnp.float32),
        pltpu.SemaphoreType.DMA,
    ]

    @functools.partial(pl.kernel, mesh=_mesh(), out_type=out_type,
                       scratch_types=scratch,
                       compiler_params=pltpu.CompilerParams(
                           use_tc_tiling_on_sc=False))
    def k(ub_ref, pb_ref, nb_ref,
          c0u0, c0u1, c0i0, c0i1, d0u0, d0u1, d0i0, d0i1,
          c1u0, c1u1, c1i0, c1i1, d1u0, d1u1, d1i0, d1i1,
          eu0_ref, eu1_ref, ei0_ref, ei1_ref,
          ps0, ns0, pc0, nc0, ps1, ns1, pc1, nc1, squ, sqi,
          uix, pix, nix, bu0, bu1, bp0, bp1, bn0, bn1,
          psb, nsb, rbuf, vbuf, sem):
        c = lax.axis_index("c")
        s = lax.axis_index("s")
        w = s * 2 + c

        pltpu.sync_copy(ub_ref.at[w], uix)
        pltpu.sync_copy(pb_ref.at[w], pix)
        pltpu.sync_copy(nb_ref.at[w], nix)

        def do_pair(tu0, tu1, ti0, ti1, pout, nout):
            descs = [
                pltpu.async_copy(tu0.at[uix], bu0, sem),
                pltpu.async_copy(tu1.at[uix], bu1, sem),
                pltpu.async_copy(ti0.at[pix], bp0, sem),
                pltpu.async_copy(ti1.at[pix], bp1, sem),
                pltpu.async_copy(ti0.at[nix], bn0, sem),
                pltpu.async_copy(ti1.at[nix], bn1, sem),
            ]
            for dsc in descs:
                dsc.wait()

            def rows(kk, carry):
                lane = lax.iota(jnp.int32, 16)
                ps16 = jnp.zeros((16,), jnp.float32)
                ns16 = jnp.zeros((16,), jnp.float32)
                for r in range(16):
                    row = kk * 16 + r
                    ps = _lanesum(bu0[row, :] * bp0[row, :]
                                  + bu1[row, :] * bp1[row, :])
                    ns = _lanesum(bu0[row, :] * bn0[row, :]
                                  + bu1[row, :] * bn1[row, :])
                    ps16 = jnp.where(lane == r, ps, ps16)
                    ns16 = jnp.where(lane == r, ns, ns16)
                psb[pl.ds(kk * 16, 16)] = ps16
                nsb[pl.ds(kk * 16, 16)] = ns16
                return carry
            lax.fori_loop(0, IW // 16, rows, 0)
            pltpu.sync_copy(psb, pout.at[pl.ds(w * IW, IW)])
            pltpu.sync_copy(nsb, nout.at[pl.ds(w * IW, IW)])

        do_pair(c0u0, c0u1, c0i0, c0i1, ps0, ns0)
        do_pair(d0u0, d0u1, d0i0, d0i1, pc0, nc0)
        do_pair(c1u0, c1u1, c1i0, c1i1, ps1, ns1)
        do_pair(d1u0, d1u1, d1i0, d1i1, pc1, nc1)

        def ssq(t0, t1, out_ref):
            accv = jnp.zeros((16,), jnp.float32)
            for b in range(NSSB):
                blk = pl.ds(w * SSB + b * SSBLK, SSBLK)

                def rw(r, a):
                    v = rbuf[r, :]
                    return a + v * v
                pltpu.sync_copy(t0.at[blk], rbuf)
                accv = lax.fori_loop(0, SSBLK, rw, accv)
                pltpu.sync_copy(t1.at[blk], rbuf)
                accv = lax.fori_loop(0, SSBLK, rw, accv)
            vbuf[:] = accv
            pltpu.sync_copy(vbuf, out_ref.at[w])

        ssq(eu0_ref, eu1_ref, squ)
        ssq(ei0_ref, ei1_ref, sqi)

    return k(ub, pb, nb, *flat_accs, eu0, eu1, ei0, ei1)


def kernel(batch_data, user_table, item_table, conf_edges_0, conf_edges_1,
           cond_edges_0, cond_edges_1):
    def pad_rows(t):
        return jnp.concatenate(
            [t, jnp.zeros((NPAD - t.shape[0], t.shape[1]), t.dtype)], axis=0)

    ut = pad_rows(user_table)
    it = pad_rows(item_table)
    eu0, eu1 = ut[:, :H], ut[:, H:]
    ei0, ei1 = it[:, :H], it[:, H:]

    def prep_edges(e):
        n = e.shape[1]
        epad = ((n + 16383) // 16384) * 16384
        z = jnp.zeros((epad - n,), jnp.int32)
        u2 = jnp.concatenate([e[0], z]).reshape(epad // IW, IW)
        i2 = jnp.concatenate([e[1], z]).reshape(epad // IW, IW)
        return u2, i2

    accs = []
    for conf_e, cond_e in ((conf_edges_0, cond_edges_0),
                           (conf_edges_1, cond_edges_1)):
        pair = []
        for e in (conf_e, cond_e):
            u2, i2 = prep_edges(e)
            dvu, dvi, tu0, tu1, ti0, ti1 = _deg_prep(
                u2, i2, eu0, eu1, ei0, ei1)
            (xu0, xu1, xi0, xi1,
             nu0, nu1, ni0, ni1) = _layer(
                u2, i2, (tu0, tu1), (ti0, ti1), dvu, dvi, mode=1)
            a = _layer(u2, i2, (nu0, nu1), (ni0, ni1), dvu, dvi, mode=2,
                       extra=(eu0, eu1, ei0, ei1, xu0, xu1, xi0, xi1))
            pair.append(a)
        accs.append(pair)

    ub = batch_data[:, 0].reshape(32, IW)
    pb = batch_data[:, 1].reshape(32, IW)
    nb = batch_data[:, 2].reshape(32, IW)
    (ps0, ns0, pc0, nc0, ps1, ns1, pc1, nc1,
     squ, sqi) = _score(ub, pb, nb, accs, eu0, eu1, ei0, ei1)

    def bpr(p, n):
        return -jnp.mean(jax.nn.log_sigmoid(p - n))

    aux_loss = (bpr(ps0, ns0) + bpr(ps1, ns1)) * 0.5
    tp_conf = jnp.stack([jax.nn.relu(ps0), jax.nn.relu(ps1)])
    tn_conf = jnp.stack([jax.nn.relu(ns0), jax.nn.relu(ns1)])
    tp_cond = jnp.stack([jax.nn.relu(pc0), jax.nn.relu(pc1)])
    tn_cond = jnp.stack([jax.nn.relu(nc0), jax.nn.relu(nc1)])

    def mmn(t):
        mn = t.min(axis=0, keepdims=True)
        mx = t.max(axis=0, keepdims=True)
        sc = (t - mn) / (mx - mn + 1e-08)
        return sc / (sc.sum(axis=0, keepdims=True) + 1e-08)

    rec_p = jnp.sum(tp_cond * mmn(tp_conf), axis=0)
    rec_n = jnp.sum(tn_cond * mmn(tn_conf), axis=0)
    rec_loss = bpr(rec_p, rec_n)
    emb_loss = (jnp.sqrt(squ.sum()) + jnp.sqrt(sqi.sum())) / item_table.shape[0]
    return rec_loss + ALPHA * aux_loss + REG_WEIGHT * emb_loss


# speedup vs baseline: 36.7733x; 1.1441x over previous
import functools

import jax
import jax.numpy as jnp
from jax import lax
from jax.experimental import pallas as pl
from jax.experimental.pallas import tpu as pltpu
from jax.experimental.pallas import tpu_sc as plsc

H = 16
NROWS = 100001
NPAD = 100352
PT = NPAD // 16
BLK = 896
NBLK = PT // BLK
LBLK = 224
LNBLK = PT // LBLK
IW = 128
GR = 8
ALPHA = 0.5
REG_WEIGHT = 0.001


def _mesh():
    return plsc.VectorSubcoreMesh(core_axis_name="c", subcore_axis_name="s")


def _f32(shape):
    return jax.ShapeDtypeStruct(shape, jnp.float32)


def _zero_rows(ref, n):
    def body(r, carry):
        ref[r, :] = jnp.zeros((H,), jnp.float32)
        return carry
    lax.fori_loop(0, n, body, 0)


def _lanesum(v):
    s = v[0]
    for r in range(1, 16):
        s = s + v[r]
    return s


def _rsqrt16(d):
    y = 1.0 / d
    for _ in range(20):
        y = y * (1.5 - 0.5 * d * y * y)
    return y


def _deg_prep(u2d, i2d, eu0, eu1, ei0, ei1):
    er = u2d.shape[0]
    cnt_r = er // 16
    ngr, rem = divmod(cnt_r, GR)

    out_type = [_f32((NPAD,))] * 2 + [_f32((NPAD, H))] * 4
    scratch = [
        pltpu.VMEM_SHARED((NPAD,), jnp.float32),
        pltpu.VMEM((GR, IW), jnp.int32),
        pltpu.VMEM((IW,), jnp.float32),
        pltpu.VMEM((BLK,), jnp.float32),
        pltpu.VMEM((BLK,), jnp.float32),
        pltpu.VMEM((BLK, H), jnp.float32),
        pltpu.VMEM((BLK, H), jnp.float32),
        pltpu.VMEM((BLK, H), jnp.float32),
        pltpu.VMEM((BLK, H), jnp.float32),
    ]

    @functools.partial(pl.kernel, mesh=_mesh(), out_type=out_type,
                       scratch_types=scratch,
                       compiler_params=pltpu.CompilerParams(
                           use_tc_tiling_on_sc=False))
    def k(u_ref, i_ref, eu0_ref, eu1_ref, ei0_ref, ei1_ref,
          dvu_ref, dvi_ref, tu0_ref, tu1_ref, ti0_ref, ti1_ref,
          deg_sh, idxb, onesb, degb, dvb, e0b, e1b, t0b, t1b):
        c = lax.axis_index("c")
        s = lax.axis_index("s")
        tb = s * PT

        for kk in range(IW // 16):
            onesb[pl.ds(kk * 16, 16)] = jnp.full((16,), 1.0, jnp.float32)

        def zb(kk, carry):
            degb[pl.ds(kk * 16, 16)] = jnp.zeros((16,), jnp.float32)
            return carry
        lax.fori_loop(0, BLK // 16, zb, 0)
        for b in range(NBLK):
            pltpu.sync_copy(degb, deg_sh.at[pl.ds(tb + b * BLK, BLK)])
        plsc.subcore_barrier()

        def scatter_side(idx2d):
            row0 = s * cnt_r

            def grp(g, carry):
                pltpu.sync_copy(idx2d.at[pl.ds(row0 + g * GR, GR)], idxb)
                for j in range(GR):
                    pltpu.sync_copy(onesb, deg_sh.at[idxb.at[j]], add=True)
                return carry
            lax.fori_loop(0, ngr, grp, 0)
            if rem:
                base = row0 + ngr * GR
                pltpu.sync_copy(idx2d.at[pl.ds(base, rem)],
                                idxb.at[pl.ds(0, rem)])
                for j in range(rem):
                    pltpu.sync_copy(onesb, deg_sh.at[idxb.at[j]], add=True)

        @pl.when(c == 0)
        def _():
            scatter_side(u_ref)

        @pl.when(c == 1)
        def _():
            scatter_side(i_ref)

        plsc.subcore_barrier()

        def drain_side(e0_ref, e1_ref, dv_ref, t0_ref, t1_ref):
            def drain(b, carry):
                off = pl.multiple_of(tb + b * BLK, BLK)
                blk = pl.ds(off, BLK)
                pltpu.sync_copy(deg_sh.at[blk], degb)

                def rsq(kk, carry):
                    sl = pl.ds(kk * 16, 16)
                    dg = degb[sl]
                    d = jnp.maximum(dg, 1.0)
                    y = _rsqrt16(d)
                    dvb[sl] = jnp.where(dg > 0.0, y,
                                        jnp.zeros((16,), jnp.float32))
                    return carry
                lax.fori_loop(0, BLK // 16, rsq, 0)
                pltpu.sync_copy(dvb, dv_ref.at[blk])
                pltpu.sync_copy(e0_ref.at[blk], e0b)
                pltpu.sync_copy(e1_ref.at[blk], e1b)

                def rowm(kk, carry):
                    dv16 = dvb[pl.ds(kk * 16, 16)]
                    for r in range(16):
                        row = kk * 16 + r
                        dv = dv16[r]
                        t0b[row, :] = e0b[row, :] * dv
                        t1b[row, :] = e1b[row, :] * dv
                    return carry
                lax.fori_loop(0, BLK // 16, rowm, 0)
                pltpu.sync_copy(t0b, t0_ref.at[blk])
                pltpu.sync_copy(t1b, t1_ref.at[blk])
                return carry
            lax.fori_loop(0, NBLK, drain, 0)

        @pl.when(c == 0)
        def _():
            drain_side(eu0_ref, eu1_ref, dvu_ref, tu0_ref, tu1_ref)

        @pl.when(c == 1)
        def _():
            drain_side(ei0_ref, ei1_ref, dvi_ref, ti0_ref, ti1_ref)

    return k(u2d, i2d, eu0, eu1, ei0, ei1)


def _layer(u2d, i2d, tu, ti, dvu, dvi, mode, extra=None):
    er = u2d.shape[0]
    cnt_r = er // 16
    ngr, rem = divmod(cnt_r, GR)

    n_out = 8 if mode == 1 else 4
    out_type = [_f32((NPAD, H))] * n_out
    scratch = [
        pltpu.VMEM_SHARED((NPAD, H), jnp.float32),
        pltpu.VMEM((GR, IW), jnp.int32),
        pltpu.VMEM((GR, IW), jnp.int32),
        pltpu.VMEM((GR * IW // 2, H), jnp.float32),
        pltpu.VMEM((GR * IW // 2, H), jnp.float32),
        pltpu.VMEM((LBLK, H), jnp.float32),
        pltpu.VMEM((LBLK,), jnp.float32),
        pltpu.VMEM((LBLK, H), jnp.float32),
        pltpu.VMEM((LBLK, H), jnp.float32),
        pltpu.SemaphoreType.DMA,
        pltpu.SemaphoreType.DMA,
        pltpu.SemaphoreType.DMA,
    ]

    def body(refs):
        if mode == 1:
            (u_ref, i_ref, tu0, tu1, ti0, ti1, dvu_ref, dvi_ref,
             xu0, xu1, xi0, xi1, nu0, nu1, ni0, ni1,
             acc_sh, sidx, didx, rb0, rb1, abuf, dvb, o1b, o2b,
             semg0, semg1, sems) = refs
        else:
            (u_ref, i_ref, tu0, tu1, ti0, ti1, dvu_ref, dvi_ref,
             eu0_ref, eu1_ref, ei0_ref, ei1_ref,
             xu0, xu1, xi0, xi1,
             au0, au1, ai0, ai1,
             acc_sh, sidx, didx, rb0, rb1, abuf, dvb, o1b, o2b,
             semg0, semg1, sems) = refs

        c = lax.axis_index("c")
        s = lax.axis_index("s")
        tb = s * PT

        def edge_pass(src2d, dst2d, tsrc):
            row0 = s * cnt_r
            half = GR // 2

            def grp(g, carry):
                gbase = pl.multiple_of(row0 + g * GR, GR)
                pltpu.sync_copy(src2d.at[pl.ds(gbase, GR)], sidx)
                pltpu.sync_copy(dst2d.at[pl.ds(gbase, GR)], didx)
                ga = [pltpu.async_copy(tsrc.at[sidx.at[j]],
                                       rb0.at[pl.ds(j * IW, IW)], semg0)
                      for j in range(half)]
                gb = [pltpu.async_copy(tsrc.at[sidx.at[half + j]],
                                       rb1.at[pl.ds(j * IW, IW)], semg1)
                      for j in range(half)]
                sc = []
                for dsc in ga:
                    dsc.wait()
                for j in range(half):
                    sc.append(pltpu.async_copy(
                        rb0.at[pl.ds(j * IW, IW)],
                        acc_sh.at[didx.at[j]], sems, add=True))
                for dsc in gb:
                    dsc.wait()
                for j in range(half):
                    sc.append(pltpu.async_copy(
                        rb1.at[pl.ds(j * IW, IW)],
                        acc_sh.at[didx.at[half + j]], sems, add=True))
                for dsc in sc:
                    dsc.wait()
                return carry
            lax.fori_loop(0, ngr, grp, 0)

        def run_side(dst2d, src2d, tsrc_pair, dv_ref, outs):
            for h in range(2):
                _zero_rows(o1b, LBLK)

                def zblk(b, carry):
                    off = pl.multiple_of(tb + b * LBLK, LBLK)
                    pltpu.sync_copy(o1b, acc_sh.at[pl.ds(off, LBLK)])
                    return carry
                lax.fori_loop(0, LNBLK, zblk, 0)
                plsc.subcore_barrier()
                edge_pass(src2d, dst2d, tsrc_pair[h])
                plsc.subcore_barrier()

                if mode == 1:
                    xout, tout = outs[0][h], outs[1][h]

                    def drain(b, carry):
                        off = pl.multiple_of(tb + b * LBLK, LBLK)
                        blk = pl.ds(off, LBLK)
                        pltpu.sync_copy(acc_sh.at[blk], abuf)
                        pltpu.sync_copy(dv_ref.at[blk], dvb)

                        def rowm(kk, c2):
                            dv16 = dvb[pl.ds(kk * 16, 16)]
                            for r in range(16):
                                row = kk * 16 + r
                                dv = dv16[r]
                                x = abuf[row, :] * dv
                                o1b[row, :] = x
                                o2b[row, :] = x * dv
                            return c2
                        lax.fori_loop(0, LBLK // 16, rowm, 0)
                        pltpu.sync_copy(o1b, xout.at[blk])
                        pltpu.sync_copy(o2b, tout.at[blk])
                        return carry
                    lax.fori_loop(0, LNBLK, drain, 0)
                else:
                    e_ref, x_ref, aout = (outs[0][h], outs[1][h],
                                          outs[2][h])

                    def drain(b, carry):
                        off = pl.multiple_of(tb + b * LBLK, LBLK)
                        blk = pl.ds(off, LBLK)
                        pltpu.sync_copy(acc_sh.at[blk], abuf)
                        pltpu.sync_copy(dv_ref.at[blk], dvb)
                        pltpu.sync_copy(e_ref.at[blk], o1b)
                        pltpu.sync_copy(x_ref.at[blk], o2b)

                        def rowm(kk, c2):
                            dv16 = dvb[pl.ds(kk * 16, 16)]
                            for r in range(16):
                                row = kk * 16 + r
                                a = (o1b[row, :] + o2b[row, :]
                                     + abuf[row, :] * dv16[r]) * (1.0 / 3.0)
                                abuf[row, :] = a
                            return c2
                        lax.fori_loop(0, LBLK // 16, rowm, 0)
                        pltpu.sync_copy(abuf, aout.at[blk])
                        return carry
                    lax.fori_loop(0, LNBLK, drain, 0)

        if mode == 1:
            @pl.when(c == 0)
            def _():
                run_side(u_ref, i_ref, (ti0, ti1), dvu_ref,
                         ((xu0, xu1), (nu0, nu1)))

            @pl.when(c == 1)
            def _():
                run_side(i_ref, u_ref, (tu0, tu1), dvi_ref,
                         ((xi0, xi1), (ni0, ni1)))
        else:
            @pl.when(c == 0)
            def _():
                run_side(u_ref, i_ref, (ti0, ti1), dvu_ref,
                         ((eu0_ref, eu1_ref), (xu0, xu1), (au0, au1)))

            @pl.when(c == 1)
            def _():
                run_side(i_ref, u_ref, (tu0, tu1), dvi_ref,
                         ((ei0_ref, ei1_ref), (xi0, xi1), (ai0, ai1)))

    @functools.partial(pl.kernel, mesh=_mesh(), out_type=out_type,
                       scratch_types=scratch,
                       compiler_params=pltpu.CompilerParams(
                           use_tc_tiling_on_sc=False))
    def k(*refs):
        body(refs)

    if mode == 1:
        return k(u2d, i2d, tu[0], tu[1], ti[0], ti[1], dvu, dvi)
    return k(u2d, i2d, tu[0], tu[1], ti[0], ti[1], dvu, dvi, *extra)


def _score(ub, pb, nb, accs, eu0, eu1, ei0, ei1):
    B = 4096
    SSB = NPAD // 32
    SSBLK = 784
    NSSB = SSB // SSBLK

    flat_accs = []
    for g in range(2):
        for kind in range(2):
            flat_accs.extend(accs[g][kind])

    out_type = [_f32((B,))] * 8 + [_f32((32, H))] * 2
    scratch = [
        pltpu.VMEM((IW,), jnp.int32),
        pltpu.VMEM((IW,), jnp.int32),
        pltpu.VMEM((IW,), jnp.int32),
        pltpu.VMEM((IW, H), jnp.float32),
        pltpu.VMEM((IW, H), jnp.float32),
        pltpu.VMEM((IW, H), jnp.float32),
        pltpu.VMEM((IW, H), jnp.float32),
        pltpu.VMEM((IW, H), jnp.float32),
        pltpu.VMEM((IW, H), jnp.float32),
        pltpu.VMEM((IW,), jnp.float32),
        pltpu.VMEM((IW,), jnp.float32),
        pltpu.VMEM((SSBLK, H), jnp.float32),
        pltpu.VMEM((16,), jnp.float32),
        pltpu.SemaphoreType.DMA,
    ]

    @functools.partial(pl.kernel, mesh=_mesh(), out_type=out_type,
                       scratch_types=scratch,
                       compiler_params=pltpu.CompilerParams(
                           use_tc_tiling_on_sc=False))
    def k(ub_ref, pb_ref, nb_ref,
          c0u0, c0u1, c0i0, c0i1, d0u0, d0u1, d0i0, d0i1,
          c1u0, c1u1, c1i0, c1i1, d1u0, d1u1, d1i0, d1i1,
          eu0_ref, eu1_ref, ei0_ref, ei1_ref,
          ps0, ns0, pc0, nc0, ps1, ns1, pc1, nc1, squ, sqi,
          uix, pix, nix, bu0, bu1, bp0, bp1, bn0, bn1,
          psb, nsb, rbuf, vbuf, sem):
        c = lax.axis_index("c")
        s = lax.axis_index("s")
        w = s * 2 + c

        pltpu.sync_copy(ub_ref.at[w], uix)
        pltpu.sync_copy(pb_ref.at[w], pix)
        pltpu.sync_copy(nb_ref.at[w], nix)

        def do_pair(tu0, tu1, ti0, ti1, pout, nout):
            descs = [
                pltpu.async_copy(tu0.at[uix], bu0, sem),
                pltpu.async_copy(tu1.at[uix], bu1, sem),
                pltpu.async_copy(ti0.at[pix], bp0, sem),
                pltpu.async_copy(ti1.at[pix], bp1, sem),
                pltpu.async_copy(ti0.at[nix], bn0, sem),
                pltpu.async_copy(ti1.at[nix], bn1, sem),
            ]
            for dsc in descs:
                dsc.wait()

            def rows(kk, carry):
                lane = lax.iota(jnp.int32, 16)
                ps16 = jnp.zeros((16,), jnp.float32)
                ns16 = jnp.zeros((16,), jnp.float32)
                for r in range(16):
                    row = kk * 16 + r
                    ps = _lanesum(bu0[row, :] * bp0[row, :]
                                  + bu1[row, :] * bp1[row, :])
                    ns = _lanesum(bu0[row, :] * bn0[row, :]
                                  + bu1[row, :] * bn1[row, :])
                    ps16 = jnp.where(lane == r, ps, ps16)
                    ns16 = jnp.where(lane == r, ns, ns16)
                psb[pl.ds(kk * 16, 16)] = ps16
                nsb[pl.ds(kk * 16, 16)] = ns16
                return carry
            lax.fori_loop(0, IW // 16, rows, 0)
            pltpu.sync_copy(psb, pout.at[pl.ds(w * IW, IW)])
            pltpu.sync_copy(nsb, nout.at[pl.ds(w * IW, IW)])

        do_pair(c0u0, c0u1, c0i0, c0i1, ps0, ns0)
        do_pair(d0u0, d0u1, d0i0, d0i1, pc0, nc0)
        do_pair(c1u0, c1u1, c1i0, c1i1, ps1, ns1)
        do_pair(d1u0, d1u1, d1i0, d1i1, pc1, nc1)

        def ssq(t0, t1, out_ref):
            accv = jnp.zeros((16,), jnp.float32)
            for b in range(NSSB):
                blk = pl.ds(w * SSB + b * SSBLK, SSBLK)

                def rw(r, a):
                    v = rbuf[r, :]
                    return a + v * v
                pltpu.sync_copy(t0.at[blk], rbuf)
                accv = lax.fori_loop(0, SSBLK, rw, accv)
                pltpu.sync_copy(t1.at[blk], rbuf)
                accv = lax.fori_loop(0, SSBLK, rw, accv)
            vbuf[:] = accv
            pltpu.sync_copy(vbuf, out_ref.at[w])

        ssq(eu0_ref, eu1_ref, squ)
        ssq(ei0_ref, ei1_ref, sqi)

    return k(ub, pb, nb, *flat_accs, eu0, eu1, ei0, ei1)


def kernel(batch_data, user_table, item_table, conf_edges_0, conf_edges_1,
           cond_edges_0, cond_edges_1):
    def pad_rows(t):
        return jnp.concatenate(
            [t, jnp.zeros((NPAD - t.shape[0], t.shape[1]), t.dtype)], axis=0)

    ut = pad_rows(user_table)
    it = pad_rows(item_table)
    eu0, eu1 = ut[:, :H], ut[:, H:]
    ei0, ei1 = it[:, :H], it[:, H:]

    def prep_edges(e):
        n = e.shape[1]
        epad = ((n + 16383) // 16384) * 16384
        z = jnp.zeros((epad - n,), jnp.int32)
        u2 = jnp.concatenate([e[0], z]).reshape(epad // IW, IW)
        i2 = jnp.concatenate([e[1], z]).reshape(epad // IW, IW)
        return u2, i2

    accs = []
    for conf_e, cond_e in ((conf_edges_0, cond_edges_0),
                           (conf_edges_1, cond_edges_1)):
        pair = []
        for e in (conf_e, cond_e):
            u2, i2 = prep_edges(e)
            dvu, dvi, tu0, tu1, ti0, ti1 = _deg_prep(
                u2, i2, eu0, eu1, ei0, ei1)
            (xu0, xu1, xi0, xi1,
             nu0, nu1, ni0, ni1) = _layer(
                u2, i2, (tu0, tu1), (ti0, ti1), dvu, dvi, mode=1)
            a = _layer(u2, i2, (nu0, nu1), (ni0, ni1), dvu, dvi, mode=2,
                       extra=(eu0, eu1, ei0, ei1, xu0, xu1, xi0, xi1))
            pair.append(a)
        accs.append(pair)

    ub = batch_data[:, 0].reshape(32, IW)
    pb = batch_data[:, 1].reshape(32, IW)
    nb = batch_data[:, 2].reshape(32, IW)
    (ps0, ns0, pc0, nc0, ps1, ns1, pc1, nc1,
     squ, sqi) = _score(ub, pb, nb, accs, eu0, eu1, ei0, ei1)

    def bpr(p, n):
        return -jnp.mean(jax.nn.log_sigmoid(p - n))

    aux_loss = (bpr(ps0, ns0) + bpr(ps1, ns1)) * 0.5
    tp_conf = jnp.stack([jax.nn.relu(ps0), jax.nn.relu(ps1)])
    tn_conf = jnp.stack([jax.nn.relu(ns0), jax.nn.relu(ns1)])
    tp_cond = jnp.stack([jax.nn.relu(pc0), jax.nn.relu(pc1)])
    tn_cond = jnp.stack([jax.nn.relu(nc0), jax.nn.relu(nc1)])

    def mmn(t):
        mn = t.min(axis=0, keepdims=True)
        mx = t.max(axis=0, keepdims=True)
        sc = (t - mn) / (mx - mn + 1e-08)
        return sc / (sc.sum(axis=0, keepdims=True) + 1e-08)

    rec_p = jnp.sum(tp_cond * mmn(tp_conf), axis=0)
    rec_n = jnp.sum(tn_cond * mmn(tn_conf), axis=0)
    rec_loss = bpr(rec_p, rec_n)
    emb_loss = (jnp.sqrt(squ.sum()) + jnp.sqrt(sqi.sum())) / item_table.shape[0]
    return rec_loss + ALPHA * aux_loss + REG_WEIGHT * emb_loss


# speedup vs baseline: 39.9372x; 1.0860x over previous
import functools

import jax
import jax.numpy as jnp
from jax import lax
from jax.experimental import pallas as pl
from jax.experimental.pallas import tpu as pltpu
from jax.experimental.pallas import tpu_sc as plsc

H = 16
NROWS = 100001
NPAD = 100352
PT = NPAD // 16
BLK = 896
NBLK = PT // BLK
LBLK = 224
LNBLK = PT // LBLK
IW = 128
GR = 8
ALPHA = 0.5
REG_WEIGHT = 0.001


def _mesh():
    return plsc.VectorSubcoreMesh(core_axis_name="c", subcore_axis_name="s")


def _f32(shape):
    return jax.ShapeDtypeStruct(shape, jnp.float32)


def _zero_rows(ref, n):
    def body(r, carry):
        ref[r, :] = jnp.zeros((H,), jnp.float32)
        return carry
    lax.fori_loop(0, n, body, 0)


def _lanesum(v):
    s = v[0]
    for r in range(1, 16):
        s = s + v[r]
    return s


def _rsqrt16(d):
    y = 1.0 / d
    for _ in range(20):
        y = y * (1.5 - 0.5 * d * y * y)
    return y


def _deg_prep(e2d, eu0, eu1, ei0, ei1):
    er = e2d.shape[0] // 2
    cnt_r = er // 16
    ngr = cnt_r // GR

    out_type = [_f32((NPAD,))] * 2 + [_f32((NPAD, H))] * 4
    scratch = [
        pltpu.VMEM_SHARED((NPAD,), jnp.float32),
        pltpu.VMEM((2 * GR, IW), jnp.int32),
        pltpu.VMEM((IW,), jnp.float32),
        pltpu.SemaphoreType.DMA,
        pltpu.VMEM((BLK,), jnp.float32),
        pltpu.VMEM((BLK,), jnp.float32),
        pltpu.VMEM((BLK, H), jnp.float32),
        pltpu.VMEM((BLK, H), jnp.float32),
        pltpu.VMEM((BLK, H), jnp.float32),
        pltpu.VMEM((BLK, H), jnp.float32),
    ]

    @functools.partial(pl.kernel, mesh=_mesh(), out_type=out_type,
                       scratch_types=scratch,
                       compiler_params=pltpu.CompilerParams(
                           use_tc_tiling_on_sc=False))
    def k(e_ref, eu0_ref, eu1_ref, ei0_ref, ei1_ref,
          dvu_ref, dvi_ref, tu0_ref, tu1_ref, ti0_ref, ti1_ref,
          deg_sh, idxb, onesb, semd, degb, dvb, e0b, e1b, t0b, t1b):
        c = lax.axis_index("c")
        s = lax.axis_index("s")
        tb = s * PT

        for kk in range(IW // 16):
            onesb[pl.ds(kk * 16, 16)] = jnp.full((16,), 1.0, jnp.float32)

        def zb(kk, carry):
            degb[pl.ds(kk * 16, 16)] = jnp.zeros((16,), jnp.float32)
            return carry
        lax.fori_loop(0, BLK // 16, zb, 0)
        for b in range(NBLK):
            pltpu.sync_copy(degb, deg_sh.at[pl.ds(tb + b * BLK, BLK)])
        plsc.subcore_barrier()

        def scatter_side(sel):
            row0 = s * cnt_r

            def grp(g, carry):
                gbase = pl.multiple_of(2 * (row0 + g * GR), 2 * GR)
                pltpu.sync_copy(e_ref.at[pl.ds(gbase, 2 * GR)], idxb)
                descs = []
                for j in range(GR):
                    descs.append(pltpu.async_copy(
                        onesb, deg_sh.at[idxb.at[2 * j + sel]], semd,
                        add=True))
                for dsc in descs:
                    dsc.wait()
                return carry
            lax.fori_loop(0, ngr, grp, 0)

        @pl.when(c == 0)
        def _():
            scatter_side(0)

        @pl.when(c == 1)
        def _():
            scatter_side(1)

        plsc.subcore_barrier()

        def drain_side(e0_ref, e1_ref, dv_ref, t0_ref, t1_ref):
            def drain(b, carry):
                off = pl.multiple_of(tb + b * BLK, BLK)
                blk = pl.ds(off, BLK)
                pltpu.sync_copy(deg_sh.at[blk], degb)

                def rsq(kk, carry):
                    sl = pl.ds(kk * 16, 16)
                    dg = degb[sl]
                    d = jnp.maximum(dg, 1.0)
                    y = _rsqrt16(d)
                    dvb[sl] = jnp.where(dg > 0.0, y,
                                        jnp.zeros((16,), jnp.float32))
                    return carry
                lax.fori_loop(0, BLK // 16, rsq, 0)
                pltpu.sync_copy(dvb, dv_ref.at[blk])
                pltpu.sync_copy(e0_ref.at[blk], e0b)
                pltpu.sync_copy(e1_ref.at[blk], e1b)

                def rowm(kk, carry):
                    dv16 = dvb[pl.ds(kk * 16, 16)]
                    for r in range(16):
                        row = kk * 16 + r
                        dv = dv16[r]
                        t0b[row, :] = e0b[row, :] * dv
                        t1b[row, :] = e1b[row, :] * dv
                    return carry
                lax.fori_loop(0, BLK // 16, rowm, 0)
                pltpu.sync_copy(t0b, t0_ref.at[blk])
                pltpu.sync_copy(t1b, t1_ref.at[blk])
                return carry
            lax.fori_loop(0, NBLK, drain, 0)

        @pl.when(c == 0)
        def _():
            drain_side(eu0_ref, eu1_ref, dvu_ref, tu0_ref, tu1_ref)

        @pl.when(c == 1)
        def _():
            drain_side(ei0_ref, ei1_ref, dvi_ref, ti0_ref, ti1_ref)

    return k(e2d, eu0, eu1, ei0, ei1)


def _layer(e2d, tu, ti, dvu, dvi, mode, extra=None):
    er = e2d.shape[0] // 2
    cnt_r = er // 16
    ngr = cnt_r // GR

    n_out = 8 if mode == 1 else 4
    out_type = [_f32((NPAD, H))] * n_out
    scratch = [
        pltpu.VMEM_SHARED((NPAD, H), jnp.float32),
        pltpu.VMEM((2 * GR, IW), jnp.int32),
        pltpu.VMEM((GR * IW // 2, H), jnp.float32),
        pltpu.VMEM((GR * IW // 2, H), jnp.float32),
        pltpu.VMEM((LBLK, H), jnp.float32),
        pltpu.VMEM((LBLK,), jnp.float32),
        pltpu.VMEM((LBLK, H), jnp.float32),
        pltpu.VMEM((LBLK, H), jnp.float32),
        pltpu.SemaphoreType.DMA,
        pltpu.SemaphoreType.DMA,
        pltpu.SemaphoreType.DMA,
    ]

    def body(refs):
        if mode == 1:
            (e_ref, tu0, tu1, ti0, ti1, dvu_ref, dvi_ref,
             xu0, xu1, xi0, xi1, nu0, nu1, ni0, ni1,
             acc_sh, eidx, rb0, rb1, abuf, dvb, o1b, o2b,
             semg0, semg1, sems) = refs
        else:
            (e_ref, tu0, tu1, ti0, ti1, dvu_ref, dvi_ref,
             eu0_ref, eu1_ref, ei0_ref, ei1_ref,
             xu0, xu1, xi0, xi1,
             au0, au1, ai0, ai1,
             acc_sh, eidx, rb0, rb1, abuf, dvb, o1b, o2b,
             semg0, semg1, sems) = refs

        c = lax.axis_index("c")
        s = lax.axis_index("s")
        tb = s * PT

        def edge_pass(sel_src, sel_dst, tsrc):
            row0 = s * cnt_r
            half = GR // 2

            def grp(g, carry):
                gbase = pl.multiple_of(2 * (row0 + g * GR), 2 * GR)
                pltpu.sync_copy(e_ref.at[pl.ds(gbase, 2 * GR)], eidx)
                ga = [pltpu.async_copy(tsrc.at[eidx.at[2 * j + sel_src]],
                                       rb0.at[pl.ds(j * IW, IW)], semg0)
                      for j in range(half)]
                gb = [pltpu.async_copy(
                          tsrc.at[eidx.at[2 * (half + j) + sel_src]],
                          rb1.at[pl.ds(j * IW, IW)], semg1)
                      for j in range(half)]
                sc = []
                for dsc in ga:
                    dsc.wait()
                for j in range(half):
                    sc.append(pltpu.async_copy(
                        rb0.at[pl.ds(j * IW, IW)],
                        acc_sh.at[eidx.at[2 * j + sel_dst]], sems,
                        add=True))
                for dsc in gb:
                    dsc.wait()
                for j in range(half):
                    sc.append(pltpu.async_copy(
                        rb1.at[pl.ds(j * IW, IW)],
                        acc_sh.at[eidx.at[2 * (half + j) + sel_dst]], sems,
                        add=True))
                for dsc in sc:
                    dsc.wait()
                return carry
            lax.fori_loop(0, ngr, grp, 0)

        def run_side(sel_dst, sel_src, tsrc_pair, dv_ref, outs):
            for h in range(2):
                _zero_rows(o1b, LBLK)

                def zblk(b, carry):
                    off = pl.multiple_of(tb + b * LBLK, LBLK)
                    pltpu.sync_copy(o1b, acc_sh.at[pl.ds(off, LBLK)])
                    return carry
                lax.fori_loop(0, LNBLK, zblk, 0)
                plsc.subcore_barrier()
                edge_pass(sel_src, sel_dst, tsrc_pair[h])
                plsc.subcore_barrier()

                if mode == 1:
                    xout, tout = outs[0][h], outs[1][h]

                    def drain(b, carry):
                        off = pl.multiple_of(tb + b * LBLK, LBLK)
                        blk = pl.ds(off, LBLK)
                        pltpu.sync_copy(acc_sh.at[blk], abuf)
                        pltpu.sync_copy(dv_ref.at[blk], dvb)

                        def rowm(kk, c2):
                            dv16 = dvb[pl.ds(kk * 16, 16)]
                            for r in range(16):
                                row = kk * 16 + r
                                dv = dv16[r]
                                x = abuf[row, :] * dv
                                o1b[row, :] = x
                                o2b[row, :] = x * dv
                            return c2
                        lax.fori_loop(0, LBLK // 16, rowm, 0)
                        pltpu.sync_copy(o1b, xout.at[blk])
                        pltpu.sync_copy(o2b, tout.at[blk])
                        return carry
                    lax.fori_loop(0, LNBLK, drain, 0)
                else:
                    e_ref, x_ref, aout = (outs[0][h], outs[1][h],
                                          outs[2][h])

                    def drain(b, carry):
                        off = pl.multiple_of(tb + b * LBLK, LBLK)
                        blk = pl.ds(off, LBLK)
                        pltpu.sync_copy(acc_sh.at[blk], abuf)
                        pltpu.sync_copy(dv_ref.at[blk], dvb)
                        pltpu.sync_copy(e_ref.at[blk], o1b)
                        pltpu.sync_copy(x_ref.at[blk], o2b)

                        def rowm(kk, c2):
                            dv16 = dvb[pl.ds(kk * 16, 16)]
                            for r in range(16):
                                row = kk * 16 + r
                                a = (o1b[row, :] + o2b[row, :]
                                     + abuf[row, :] * dv16[r]) * (1.0 / 3.0)
                                abuf[row, :] = a
                            return c2
                        lax.fori_loop(0, LBLK // 16, rowm, 0)
                        pltpu.sync_copy(abuf, aout.at[blk])
                        return carry
                    lax.fori_loop(0, LNBLK, drain, 0)

        if mode == 1:
            @pl.when(c == 0)
            def _():
                run_side(0, 1, (ti0, ti1), dvu_ref,
                         ((xu0, xu1), (nu0, nu1)))

            @pl.when(c == 1)
            def _():
                run_side(1, 0, (tu0, tu1), dvi_ref,
                         ((xi0, xi1), (ni0, ni1)))
        else:
            @pl.when(c == 0)
            def _():
                run_side(0, 1, (ti0, ti1), dvu_ref,
                         ((eu0_ref, eu1_ref), (xu0, xu1), (au0, au1)))

            @pl.when(c == 1)
            def _():
                run_side(1, 0, (tu0, tu1), dvi_ref,
                         ((ei0_ref, ei1_ref), (xi0, xi1), (ai0, ai1)))

    @functools.partial(pl.kernel, mesh=_mesh(), out_type=out_type,
                       scratch_types=scratch,
                       compiler_params=pltpu.CompilerParams(
                           use_tc_tiling_on_sc=False))
    def k(*refs):
        body(refs)

    if mode == 1:
        return k(e2d, tu[0], tu[1], ti[0], ti[1], dvu, dvi)
    return k(e2d, tu[0], tu[1], ti[0], ti[1], dvu, dvi, *extra)


def _score(ub, pb, nb, accs, eu0, eu1, ei0, ei1):
    B = 4096
    SSB = NPAD // 32
    SSBLK = 784
    NSSB = SSB // SSBLK

    flat_accs = []
    for g in range(2):
        for kind in range(2):
            flat_accs.extend(accs[g][kind])

    out_type = [_f32((B,))] * 8 + [_f32((32, H))] * 2
    scratch = [
        pltpu.VMEM((IW,), jnp.int32),
        pltpu.VMEM((IW,), jnp.int32),
        pltpu.VMEM((IW,), jnp.int32),
        pltpu.VMEM((IW, H), jnp.float32),
        pltpu.VMEM((IW, H), jnp.float32),
        pltpu.VMEM((IW, H), jnp.float32),
        pltpu.VMEM((IW, H), jnp.float32),
        pltpu.VMEM((IW, H), jnp.float32),
        pltpu.VMEM((IW, H), jnp.float32),
        pltpu.VMEM((IW,), jnp.float32),
        pltpu.VMEM((IW,), jnp.float32),
        pltpu.VMEM((SSBLK, H), jnp.float32),
        pltpu.VMEM((16,), jnp.float32),
        pltpu.SemaphoreType.DMA,
    ]

    @functools.partial(pl.kernel, mesh=_mesh(), out_type=out_type,
                       scratch_types=scratch,
                       compiler_params=pltpu.CompilerParams(
                           use_tc_tiling_on_sc=False))
    def k(ub_ref, pb_ref, nb_ref,
          c0u0, c0u1, c0i0, c0i1, d0u0, d0u1, d0i0, d0i1,
          c1u0, c1u1, c1i0, c1i1, d1u0, d1u1, d1i0, d1i1,
          eu0_ref, eu1_ref, ei0_ref, ei1_ref,
          ps0, ns0, pc0, nc0, ps1, ns1, pc1, nc1, squ, sqi,
          uix, pix, nix, bu0, bu1, bp0, bp1, bn0, bn1,
          psb, nsb, rbuf, vbuf, sem):
        c = lax.axis_index("c")
        s = lax.axis_index("s")
        w = s * 2 + c

        pltpu.sync_copy(ub_ref.at[w], uix)
        pltpu.sync_copy(pb_ref.at[w], pix)
        pltpu.sync_copy(nb_ref.at[w], nix)

        def do_pair(tu0, tu1, ti0, ti1, pout, nout):
            descs = [
                pltpu.async_copy(tu0.at[uix], bu0, sem),
                pltpu.async_copy(tu1.at[uix], bu1, sem),
                pltpu.async_copy(ti0.at[pix], bp0, sem),
                pltpu.async_copy(ti1.at[pix], bp1, sem),
                pltpu.async_copy(ti0.at[nix], bn0, sem),
                pltpu.async_copy(ti1.at[nix], bn1, sem),
            ]
            for dsc in descs:
                dsc.wait()

            def rows(kk, carry):
                lane = lax.iota(jnp.int32, 16)
                ps16 = jnp.zeros((16,), jnp.float32)
                ns16 = jnp.zeros((16,), jnp.float32)
                for r in range(16):
                    row = kk * 16 + r
                    ps = _lanesum(bu0[row, :] * bp0[row, :]
                                  + bu1[row, :] * bp1[row, :])
                    ns = _lanesum(bu0[row, :] * bn0[row, :]
                                  + bu1[row, :] * bn1[row, :])
                    ps16 = jnp.where(lane == r, ps, ps16)
                    ns16 = jnp.where(lane == r, ns, ns16)
                psb[pl.ds(kk * 16, 16)] = ps16
                nsb[pl.ds(kk * 16, 16)] = ns16
                return carry
            lax.fori_loop(0, IW // 16, rows, 0)
            pltpu.sync_copy(psb, pout.at[pl.ds(w * IW, IW)])
            pltpu.sync_copy(nsb, nout.at[pl.ds(w * IW, IW)])

        do_pair(c0u0, c0u1, c0i0, c0i1, ps0, ns0)
        do_pair(d0u0, d0u1, d0i0, d0i1, pc0, nc0)
        do_pair(c1u0, c1u1, c1i0, c1i1, ps1, ns1)
        do_pair(d1u0, d1u1, d1i0, d1i1, pc1, nc1)

        def ssq(t0, t1, out_ref):
            accv = jnp.zeros((16,), jnp.float32)
            for b in range(NSSB):
                blk = pl.ds(w * SSB + b * SSBLK, SSBLK)

                def rw(r, a):
                    v = rbuf[r, :]
                    return a + v * v
                pltpu.sync_copy(t0.at[blk], rbuf)
                accv = lax.fori_loop(0, SSBLK, rw, accv)
                pltpu.sync_copy(t1.at[blk], rbuf)
                accv = lax.fori_loop(0, SSBLK, rw, accv)
            vbuf[:] = accv
            pltpu.sync_copy(vbuf, out_ref.at[w])

        ssq(eu0_ref, eu1_ref, squ)
        ssq(ei0_ref, ei1_ref, sqi)

    return k(ub, pb, nb, *flat_accs, eu0, eu1, ei0, ei1)


def kernel(batch_data, user_table, item_table, conf_edges_0, conf_edges_1,
           cond_edges_0, cond_edges_1):
    def pad_rows(t):
        return jnp.concatenate(
            [t, jnp.zeros((NPAD - t.shape[0], t.shape[1]), t.dtype)], axis=0)

    ut = pad_rows(user_table)
    it = pad_rows(item_table)
    eu0, eu1 = ut[:, :H], ut[:, H:]
    ei0, ei1 = it[:, :H], it[:, H:]

    def prep_edges(e):
        n = e.shape[1]
        epad = ((n + 16383) // 16384) * 16384
        z = jnp.zeros((epad - n,), jnp.int32)
        u2 = jnp.concatenate([e[0], z]).reshape(epad // IW, IW)
        i2 = jnp.concatenate([e[1], z]).reshape(epad // IW, IW)
        return jnp.stack([u2, i2], axis=1).reshape(2 * (epad // IW), IW)

    accs = []
    for conf_e, cond_e in ((conf_edges_0, cond_edges_0),
                           (conf_edges_1, cond_edges_1)):
        pair = []
        for e in (conf_e, cond_e):
            e2 = prep_edges(e)
            dvu, dvi, tu0, tu1, ti0, ti1 = _deg_prep(
                e2, eu0, eu1, ei0, ei1)
            (xu0, xu1, xi0, xi1,
             nu0, nu1, ni0, ni1) = _layer(
                e2, (tu0, tu1), (ti0, ti1), dvu, dvi, mode=1)
            a = _layer(e2, (nu0, nu1), (ni0, ni1), dvu, dvi, mode=2,
                       extra=(eu0, eu1, ei0, ei1, xu0, xu1, xi0, xi1))
            pair.append(a)
        accs.append(pair)

    ub = batch_data[:, 0].reshape(32, IW)
    pb = batch_data[:, 1].reshape(32, IW)
    nb = batch_data[:, 2].reshape(32, IW)
    (ps0, ns0, pc0, nc0, ps1, ns1, pc1, nc1,
     squ, sqi) = _score(ub, pb, nb, accs, eu0, eu1, ei0, ei1)

    def bpr(p, n):
        return -jnp.mean(jax.nn.log_sigmoid(p - n))

    aux_loss = (bpr(ps0, ns0) + bpr(ps1, ns1)) * 0.5
    tp_conf = jnp.stack([jax.nn.relu(ps0), jax.nn.relu(ps1)])
    tn_conf = jnp.stack([jax.nn.relu(ns0), jax.nn.relu(ns1)])
    tp_cond = jnp.stack([jax.nn.relu(pc0), jax.nn.relu(pc1)])
    tn_cond = jnp.stack([jax.nn.relu(nc0), jax.nn.relu(nc1)])

    def mmn(t):
        mn = t.min(axis=0, keepdims=True)
        mx = t.max(axis=0, keepdims=True)
        sc = (t - mn) / (mx - mn + 1e-08)
        return sc / (sc.sum(axis=0, keepdims=True) + 1e-08)

    rec_p = jnp.sum(tp_cond * mmn(tp_conf), axis=0)
    rec_n = jnp.sum(tn_cond * mmn(tn_conf), axis=0)
    rec_loss = bpr(rec_p, rec_n)
    emb_loss = (jnp.sqrt(squ.sum()) + jnp.sqrt(sqi.sum())) / item_table.shape[0]
    return rec_loss + ALPHA * aux_loss + REG_WEIGHT * emb_loss


# speedup vs baseline: 40.0058x; 1.0017x over previous
import functools

import jax
import jax.numpy as jnp
from jax import lax
from jax.experimental import pallas as pl
from jax.experimental.pallas import tpu as pltpu
from jax.experimental.pallas import tpu_sc as plsc

H = 16
NROWS = 100001
NPAD = 100352
PT = NPAD // 16
BLK = 896
NBLK = PT // BLK
LBLK = 224
LNBLK = PT // LBLK
IW = 128
GR = 8
ALPHA = 0.5
REG_WEIGHT = 0.001


def _mesh():
    return plsc.VectorSubcoreMesh(core_axis_name="c", subcore_axis_name="s")


def _f32(shape):
    return jax.ShapeDtypeStruct(shape, jnp.float32)


def _zero_rows(ref, n):
    def body(r, carry):
        ref[r, :] = jnp.zeros((H,), jnp.float32)
        return carry
    lax.fori_loop(0, n, body, 0)


def _lanesum(v):
    s = v[0]
    for r in range(1, 16):
        s = s + v[r]
    return s


def _rsqrt16(d):
    y = 1.0 / d
    for _ in range(20):
        y = y * (1.5 - 0.5 * d * y * y)
    return y


def _deg_prep(e2d, eu0, eu1, ei0, ei1):
    er = e2d.shape[0] // 2
    cnt_r = er // 16
    ngr = cnt_r // GR

    out_type = [_f32((NPAD,))] * 2 + [_f32((NPAD, H))] * 4
    scratch = [
        pltpu.VMEM_SHARED((NPAD,), jnp.float32),
        pltpu.VMEM((2 * GR, IW), jnp.int32),
        pltpu.VMEM((IW,), jnp.float32),
        pltpu.SemaphoreType.DMA,
        pltpu.VMEM((BLK,), jnp.float32),
        pltpu.VMEM((BLK,), jnp.float32),
        pltpu.VMEM((BLK, H), jnp.float32),
        pltpu.VMEM((BLK, H), jnp.float32),
        pltpu.VMEM((BLK, H), jnp.float32),
        pltpu.VMEM((BLK, H), jnp.float32),
    ]

    @functools.partial(pl.kernel, mesh=_mesh(), out_type=out_type,
                       scratch_types=scratch,
                       compiler_params=pltpu.CompilerParams(
                           use_tc_tiling_on_sc=False))
    def k(e_ref, eu0_ref, eu1_ref, ei0_ref, ei1_ref,
          dvu_ref, dvi_ref, tu0_ref, tu1_ref, ti0_ref, ti1_ref,
          deg_sh, idxb, onesb, semd, degb, dvb, e0b, e1b, t0b, t1b):
        c = lax.axis_index("c")
        s = lax.axis_index("s")
        tb = s * PT

        for kk in range(IW // 16):
            onesb[pl.ds(kk * 16, 16)] = jnp.full((16,), 1.0, jnp.float32)

        def zb(kk, carry):
            degb[pl.ds(kk * 16, 16)] = jnp.zeros((16,), jnp.float32)
            return carry
        lax.fori_loop(0, BLK // 16, zb, 0)
        for b in range(NBLK):
            pltpu.sync_copy(degb, deg_sh.at[pl.ds(tb + b * BLK, BLK)])
        plsc.subcore_barrier()

        def scatter_side(sel):
            row0 = s * cnt_r

            def grp(g, carry):
                gbase = pl.multiple_of(2 * (row0 + g * GR), 2 * GR)
                pltpu.sync_copy(e_ref.at[pl.ds(gbase, 2 * GR)], idxb)
                descs = []
                for j in range(GR):
                    descs.append(pltpu.async_copy(
                        onesb, deg_sh.at[idxb.at[2 * j + sel]], semd,
                        add=True))
                for dsc in descs:
                    dsc.wait()
                return carry
            lax.fori_loop(0, ngr, grp, 0)

        @pl.when(c == 0)
        def _():
            scatter_side(0)

        @pl.when(c == 1)
        def _():
            scatter_side(1)

        plsc.subcore_barrier()

        def drain_side(e0_ref, e1_ref, dv_ref, t0_ref, t1_ref):
            def drain(b, carry):
                off = pl.multiple_of(tb + b * BLK, BLK)
                blk = pl.ds(off, BLK)
                pltpu.sync_copy(deg_sh.at[blk], degb)

                def rsq(kk, carry):
                    sl = pl.ds(kk * 16, 16)
                    dg = degb[sl]
                    d = jnp.maximum(dg, 1.0)
                    y = _rsqrt16(d)
                    dvb[sl] = jnp.where(dg > 0.0, y,
                                        jnp.zeros((16,), jnp.float32))
                    return carry
                lax.fori_loop(0, BLK // 16, rsq, 0)
                pltpu.sync_copy(dvb, dv_ref.at[blk])
                pltpu.sync_copy(e0_ref.at[blk], e0b)
                pltpu.sync_copy(e1_ref.at[blk], e1b)

                def rowm(kk, carry):
                    dv16 = dvb[pl.ds(kk * 16, 16)]
                    for r in range(16):
                        row = kk * 16 + r
                        dv = dv16[r]
                        t0b[row, :] = e0b[row, :] * dv
                        t1b[row, :] = e1b[row, :] * dv
                    return carry
                lax.fori_loop(0, BLK // 16, rowm, 0)
                pltpu.sync_copy(t0b, t0_ref.at[blk])
                pltpu.sync_copy(t1b, t1_ref.at[blk])
                return carry
            lax.fori_loop(0, NBLK, drain, 0)

        @pl.when(c == 0)
        def _():
            drain_side(eu0_ref, eu1_ref, dvu_ref, tu0_ref, tu1_ref)

        @pl.when(c == 1)
        def _():
            drain_side(ei0_ref, ei1_ref, dvi_ref, ti0_ref, ti1_ref)

    return k(e2d, eu0, eu1, ei0, ei1)


def _layer(e2d, tu, ti, dvu, dvi, mode, extra=None):
    er = e2d.shape[0] // 2
    cnt_r = er // 16
    ngr = cnt_r // GR

    n_out = 8 if mode == 1 else 4
    out_type = [_f32((NPAD, H))] * n_out
    scratch = [
        pltpu.VMEM_SHARED((NPAD, H), jnp.float32),
        pltpu.VMEM((2 * GR, IW), jnp.int32),
        pltpu.VMEM((GR * IW // 2, H), jnp.float32),
        pltpu.VMEM((GR * IW // 2, H), jnp.float32),
        pltpu.VMEM((LBLK, H), jnp.float32),
        pltpu.VMEM((LBLK,), jnp.float32),
        pltpu.VMEM((LBLK, H), jnp.float32),
        pltpu.VMEM((LBLK, H), jnp.float32),
        pltpu.SemaphoreType.DMA,
        pltpu.SemaphoreType.DMA,
        pltpu.SemaphoreType.DMA,
    ]

    def body(refs):
        if mode == 1:
            (e_ref, tu0, tu1, ti0, ti1, dvu_ref, dvi_ref,
             xu0, xu1, xi0, xi1, nu0, nu1, ni0, ni1,
             acc_sh, eidx, rb0, rb1, abuf, dvb, o1b, o2b,
             semg0, semg1, sems) = refs
        else:
            (e_ref, tu0, tu1, ti0, ti1, dvu_ref, dvi_ref,
             eu0_ref, eu1_ref, ei0_ref, ei1_ref,
             xu0, xu1, xi0, xi1,
             au0, au1, ai0, ai1,
             acc_sh, eidx, rb0, rb1, abuf, dvb, o1b, o2b,
             semg0, semg1, sems) = refs

        c = lax.axis_index("c")
        s = lax.axis_index("s")
        tb = s * PT

        def edge_pass(sel_src, sel_dst, tsrc):
            row0 = s * cnt_r
            half = GR // 2

            def grp(g, carry):
                gbase = pl.multiple_of(2 * (row0 + g * GR), 2 * GR)
                pltpu.sync_copy(e_ref.at[pl.ds(gbase, 2 * GR)], eidx)
                ga = [pltpu.async_copy(tsrc.at[eidx.at[2 * j + sel_src]],
                                       rb0.at[pl.ds(j * IW, IW)], semg0)
                      for j in range(half)]
                gb = [pltpu.async_copy(
                          tsrc.at[eidx.at[2 * (half + j) + sel_src]],
                          rb1.at[pl.ds(j * IW, IW)], semg1)
                      for j in range(half)]
                sc = []
                for dsc in ga:
                    dsc.wait()
                for j in range(half):
                    sc.append(pltpu.async_copy(
                        rb0.at[pl.ds(j * IW, IW)],
                        acc_sh.at[eidx.at[2 * j + sel_dst]], sems,
                        add=True))
                for dsc in gb:
                    dsc.wait()
                for j in range(half):
                    sc.append(pltpu.async_copy(
                        rb1.at[pl.ds(j * IW, IW)],
                        acc_sh.at[eidx.at[2 * (half + j) + sel_dst]], sems,
                        add=True))
                for dsc in sc:
                    dsc.wait()
                return carry
            lax.fori_loop(0, ngr, grp, 0)

        def run_side(sel_dst, sel_src, tsrc_pair, dv_ref, outs):
            for h in range(2):
                _zero_rows(o1b, LBLK)
                zds = [pltpu.async_copy(
                           o1b, acc_sh.at[pl.ds(tb + b * LBLK, LBLK)], sems)
                       for b in range(LNBLK)]
                for dsc in zds:
                    dsc.wait()
                plsc.subcore_barrier()
                edge_pass(sel_src, sel_dst, tsrc_pair[h])
                plsc.subcore_barrier()

                if mode == 1:
                    xout, tout = outs[0][h], outs[1][h]

                    def drain(b, carry):
                        off = pl.multiple_of(tb + b * LBLK, LBLK)
                        blk = pl.ds(off, LBLK)
                        pltpu.sync_copy(acc_sh.at[blk], abuf)
                        pltpu.sync_copy(dv_ref.at[blk], dvb)

                        def rowm(kk, c2):
                            dv16 = dvb[pl.ds(kk * 16, 16)]
                            for r in range(16):
                                row = kk * 16 + r
                                dv = dv16[r]
                                x = abuf[row, :] * dv
                                o1b[row, :] = x
                                o2b[row, :] = x * dv
                            return c2
                        lax.fori_loop(0, LBLK // 16, rowm, 0)
                        pltpu.sync_copy(o1b, xout.at[blk])
                        pltpu.sync_copy(o2b, tout.at[blk])
                        return carry
                    lax.fori_loop(0, LNBLK, drain, 0)
                else:
                    e_ref, x_ref, aout = (outs[0][h], outs[1][h],
                                          outs[2][h])

                    def drain(b, carry):
                        off = pl.multiple_of(tb + b * LBLK, LBLK)
                        blk = pl.ds(off, LBLK)
                        pltpu.sync_copy(acc_sh.at[blk], abuf)
                        pltpu.sync_copy(dv_ref.at[blk], dvb)
                        pltpu.sync_copy(e_ref.at[blk], o1b)
                        pltpu.sync_copy(x_ref.at[blk], o2b)

                        def rowm(kk, c2):
                            dv16 = dvb[pl.ds(kk * 16, 16)]
                            for r in range(16):
                                row = kk * 16 + r
                                a = (o1b[row, :] + o2b[row, :]
                                     + abuf[row, :] * dv16[r]) * (1.0 / 3.0)
                                abuf[row, :] = a
                            return c2
                        lax.fori_loop(0, LBLK // 16, rowm, 0)
                        pltpu.sync_copy(abuf, aout.at[blk])
                        return carry
                    lax.fori_loop(0, LNBLK, drain, 0)

        if mode == 1:
            @pl.when(c == 0)
            def _():
                run_side(0, 1, (ti0, ti1), dvu_ref,
                         ((xu0, xu1), (nu0, nu1)))

            @pl.when(c == 1)
            def _():
                run_side(1, 0, (tu0, tu1), dvi_ref,
                         ((xi0, xi1), (ni0, ni1)))
        else:
            @pl.when(c == 0)
            def _():
                run_side(0, 1, (ti0, ti1), dvu_ref,
                         ((eu0_ref, eu1_ref), (xu0, xu1), (au0, au1)))

            @pl.when(c == 1)
            def _():
                run_side(1, 0, (tu0, tu1), dvi_ref,
                         ((ei0_ref, ei1_ref), (xi0, xi1), (ai0, ai1)))

    @functools.partial(pl.kernel, mesh=_mesh(), out_type=out_type,
                       scratch_types=scratch,
                       compiler_params=pltpu.CompilerParams(
                           use_tc_tiling_on_sc=False))
    def k(*refs):
        body(refs)

    if mode == 1:
        return k(e2d, tu[0], tu[1], ti[0], ti[1], dvu, dvi)
    return k(e2d, tu[0], tu[1], ti[0], ti[1], dvu, dvi, *extra)


def _score(ub, pb, nb, accs, eu0, eu1, ei0, ei1):
    B = 4096
    SSB = NPAD // 32
    SSBLK = 784
    NSSB = SSB // SSBLK

    flat_accs = []
    for g in range(2):
        for kind in range(2):
            flat_accs.extend(accs[g][kind])

    out_type = [_f32((B,))] * 8 + [_f32((32, H))] * 2
    scratch = [
        pltpu.VMEM((IW,), jnp.int32),
        pltpu.VMEM((IW,), jnp.int32),
        pltpu.VMEM((IW,), jnp.int32),
        pltpu.VMEM((IW, H), jnp.float32),
        pltpu.VMEM((IW, H), jnp.float32),
        pltpu.VMEM((IW, H), jnp.float32),
        pltpu.VMEM((IW, H), jnp.float32),
        pltpu.VMEM((IW, H), jnp.float32),
        pltpu.VMEM((IW, H), jnp.float32),
        pltpu.VMEM((IW,), jnp.float32),
        pltpu.VMEM((IW,), jnp.float32),
        pltpu.VMEM((SSBLK, H), jnp.float32),
        pltpu.VMEM((16,), jnp.float32),
        pltpu.SemaphoreType.DMA,
    ]

    @functools.partial(pl.kernel, mesh=_mesh(), out_type=out_type,
                       scratch_types=scratch,
                       compiler_params=pltpu.CompilerParams(
                           use_tc_tiling_on_sc=False))
    def k(ub_ref, pb_ref, nb_ref,
          c0u0, c0u1, c0i0, c0i1, d0u0, d0u1, d0i0, d0i1,
          c1u0, c1u1, c1i0, c1i1, d1u0, d1u1, d1i0, d1i1,
          eu0_ref, eu1_ref, ei0_ref, ei1_ref,
          ps0, ns0, pc0, nc0, ps1, ns1, pc1, nc1, squ, sqi,
          uix, pix, nix, bu0, bu1, bp0, bp1, bn0, bn1,
          psb, nsb, rbuf, vbuf, sem):
        c = lax.axis_index("c")
        s = lax.axis_index("s")
        w = s * 2 + c

        pltpu.sync_copy(ub_ref.at[w], uix)
        pltpu.sync_copy(pb_ref.at[w], pix)
        pltpu.sync_copy(nb_ref.at[w], nix)

        def do_pair(tu0, tu1, ti0, ti1, pout, nout):
            descs = [
                pltpu.async_copy(tu0.at[uix], bu0, sem),
                pltpu.async_copy(tu1.at[uix], bu1, sem),
                pltpu.async_copy(ti0.at[pix], bp0, sem),
                pltpu.async_copy(ti1.at[pix], bp1, sem),
                pltpu.async_copy(ti0.at[nix], bn0, sem),
                pltpu.async_copy(ti1.at[nix], bn1, sem),
            ]
            for dsc in descs:
                dsc.wait()

            def rows(kk, carry):
                lane = lax.iota(jnp.int32, 16)
                ps16 = jnp.zeros((16,), jnp.float32)
                ns16 = jnp.zeros((16,), jnp.float32)
                for r in range(16):
                    row = kk * 16 + r
                    ps = _lanesum(bu0[row, :] * bp0[row, :]
                                  + bu1[row, :] * bp1[row, :])
                    ns = _lanesum(bu0[row, :] * bn0[row, :]
                                  + bu1[row, :] * bn1[row, :])
                    ps16 = jnp.where(lane == r, ps, ps16)
                    ns16 = jnp.where(lane == r, ns, ns16)
                psb[pl.ds(kk * 16, 16)] = ps16
                nsb[pl.ds(kk * 16, 16)] = ns16
                return carry
            lax.fori_loop(0, IW // 16, rows, 0)
            pltpu.sync_copy(psb, pout.at[pl.ds(w * IW, IW)])
            pltpu.sync_copy(nsb, nout.at[pl.ds(w * IW, IW)])

        do_pair(c0u0, c0u1, c0i0, c0i1, ps0, ns0)
        do_pair(d0u0, d0u1, d0i0, d0i1, pc0, nc0)
        do_pair(c1u0, c1u1, c1i0, c1i1, ps1, ns1)
        do_pair(d1u0, d1u1, d1i0, d1i1, pc1, nc1)

        def ssq(t0, t1, out_ref):
            accv = jnp.zeros((16,), jnp.float32)
            for b in range(NSSB):
                blk = pl.ds(w * SSB + b * SSBLK, SSBLK)

                def rw(r, a):
                    v = rbuf[r, :]
                    return a + v * v
                pltpu.sync_copy(t0.at[blk], rbuf)
                accv = lax.fori_loop(0, SSBLK, rw, accv)
                pltpu.sync_copy(t1.at[blk], rbuf)
                accv = lax.fori_loop(0, SSBLK, rw, accv)
            vbuf[:] = accv
            pltpu.sync_copy(vbuf, out_ref.at[w])

        ssq(eu0_ref, eu1_ref, squ)
        ssq(ei0_ref, ei1_ref, sqi)

    return k(ub, pb, nb, *flat_accs, eu0, eu1, ei0, ei1)


def kernel(batch_data, user_table, item_table, conf_edges_0, conf_edges_1,
           cond_edges_0, cond_edges_1):
    def pad_rows(t):
        return jnp.concatenate(
            [t, jnp.zeros((NPAD - t.shape[0], t.shape[1]), t.dtype)], axis=0)

    ut = pad_rows(user_table)
    it = pad_rows(item_table)
    eu0, eu1 = ut[:, :H], ut[:, H:]
    ei0, ei1 = it[:, :H], it[:, H:]

    def prep_edges(e):
        n = e.shape[1]
        epad = ((n + 16383) // 16384) * 16384
        z = jnp.zeros((epad - n,), jnp.int32)
        u2 = jnp.concatenate([e[0], z]).reshape(epad // IW, IW)
        i2 = jnp.concatenate([e[1], z]).reshape(epad // IW, IW)
        return jnp.stack([u2, i2], axis=1).reshape(2 * (epad // IW), IW)

    accs = []
    for conf_e, cond_e in ((conf_edges_0, cond_edges_0),
                           (conf_edges_1, cond_edges_1)):
        pair = []
        for e in (conf_e, cond_e):
            e2 = prep_edges(e)
            dvu, dvi, tu0, tu1, ti0, ti1 = _deg_prep(
                e2, eu0, eu1, ei0, ei1)
            (xu0, xu1, xi0, xi1,
             nu0, nu1, ni0, ni1) = _layer(
                e2, (tu0, tu1), (ti0, ti1), dvu, dvi, mode=1)
            a = _layer(e2, (nu0, nu1), (ni0, ni1), dvu, dvi, mode=2,
                       extra=(eu0, eu1, ei0, ei1, xu0, xu1, xi0, xi1))
            pair.append(a)
        accs.append(pair)

    ub = batch_data[:, 0].reshape(32, IW)
    pb = batch_data[:, 1].reshape(32, IW)
    nb = batch_data[:, 2].reshape(32, IW)
    (ps0, ns0, pc0, nc0, ps1, ns1, pc1, nc1,
     squ, sqi) = _score(ub, pb, nb, accs, eu0, eu1, ei0, ei1)

    def bpr(p, n):
        return -jnp.mean(jax.nn.log_sigmoid(p - n))

    aux_loss = (bpr(ps0, ns0) + bpr(ps1, ns1)) * 0.5
    tp_conf = jnp.stack([jax.nn.relu(ps0), jax.nn.relu(ps1)])
    tn_conf = jnp.stack([jax.nn.relu(ns0), jax.nn.relu(ns1)])
    tp_cond = jnp.stack([jax.nn.relu(pc0), jax.nn.relu(pc1)])
    tn_cond = jnp.stack([jax.nn.relu(nc0), jax.nn.relu(nc1)])

    def mmn(t):
        mn = t.min(axis=0, keepdims=True)
        mx = t.max(axis=0, keepdims=True)
        sc = (t - mn) / (mx - mn + 1e-08)
        return sc / (sc.sum(axis=0, keepdims=True) + 1e-08)

    rec_p = jnp.sum(tp_cond * mmn(tp_conf), axis=0)
    rec_n = jnp.sum(tn_cond * mmn(tn_conf), axis=0)
    rec_loss = bpr(rec_p, rec_n)
    emb_loss = (jnp.sqrt(squ.sum()) + jnp.sqrt(sqi.sum())) / item_table.shape[0]
    return rec_loss + ALPHA * aux_loss + REG_WEIGHT * emb_loss
